# baseline (speedup 1.0000x reference)
import functools

import jax
import jax.numpy as jnp
from jax import lax
from jax.experimental import pallas as pl
from jax.experimental.pallas import tpu as pltpu
from jax.experimental.pallas import tpu_sc as plsc

NW = 32
TOTAL_IN = 32 * 2 * 480000
TOTAL_OUT = TOTAL_IN * 5 // 4
IN_PER_W = TOTAL_IN // NW
OUT_PER_W = TOTAL_OUT // NW
OUT_CHUNK = 60_000
IN_CHUNK = OUT_CHUNK * 4 // 5
N_CHUNKS = OUT_PER_W // OUT_CHUNK
SUPER = OUT_CHUNK // 80


def _resample_flat(xf):
    mesh = plsc.VectorSubcoreMesh(core_axis_name="c", subcore_axis_name="s")

    @functools.partial(
        pl.kernel,
        out_type=jax.ShapeDtypeStruct((TOTAL_OUT,), jnp.float32),
        mesh=mesh,
        scratch_types=[
            pltpu.VMEM((IN_CHUNK,), jnp.float32),
            pltpu.VMEM((OUT_CHUNK,), jnp.float32),
        ],
        compiler_params=pltpu.CompilerParams(
            needs_layout_passes=False,
            skip_device_barrier=True,
            disable_bounds_checks=True,
        ),
    )
    def k(x_hbm, out_hbm, in_v, out_v):
        wid = lax.axis_index("s") * 2 + lax.axis_index("c")
        in_base = wid * IN_PER_W
        out_base = wid * OUT_PER_W

        def chunk_body(c, carry):
            pltpu.sync_copy(
                x_hbm.at[pl.ds(in_base + c * IN_CHUNK, IN_CHUNK)], in_v
            )
            pltpu.sync_copy(
                out_v, out_hbm.at[pl.ds(out_base + c * OUT_CHUNK, OUT_CHUNK)]
            )
            return carry

        lax.fori_loop(0, N_CHUNKS, chunk_body, 0)

    return k(xf)


def kernel(x):
    b, ch, _ = x.shape
    out = _resample_flat(x.reshape(-1))
    return out.reshape(b, ch, TOTAL_OUT // (b * ch))


# speedup vs baseline: 1.0136x; 1.0136x over previous
import functools

import jax
import jax.numpy as jnp
from jax import lax
from jax.experimental import pallas as pl
from jax.experimental.pallas import tpu as pltpu
from jax.experimental.pallas import tpu_sc as plsc

NW = 32
TOTAL_IN = 32 * 2 * 480000
TOTAL_OUT = TOTAL_IN * 5 // 4
IN_PER_W = TOTAL_IN // NW
OUT_PER_W = TOTAL_OUT // NW
OUT_CHUNK = 60_000
IN_CHUNK = OUT_CHUNK * 4 // 5
N_CHUNKS = OUT_PER_W // OUT_CHUNK
SUPER = OUT_CHUNK // 80


def _resample_flat(xf):
    mesh = plsc.VectorSubcoreMesh(core_axis_name="c", subcore_axis_name="s")

    @functools.partial(
        pl.kernel,
        out_type=jax.ShapeDtypeStruct((TOTAL_OUT,), jnp.float32),
        mesh=mesh,
        scratch_types=[
            pltpu.VMEM((IN_CHUNK,), jnp.float32),
            pltpu.VMEM((OUT_CHUNK,), jnp.float32),
        ],
        compiler_params=pltpu.CompilerParams(
            needs_layout_passes=False,
            skip_device_barrier=True,
            disable_bounds_checks=True,
        ),
    )
    def k(x_hbm, out_hbm, in_v, out_v):
        wid = lax.axis_index("s") * 2 + lax.axis_index("c")
        in_base = wid * IN_PER_W
        out_base = wid * OUT_PER_W

        def chunk_body(c, carry):
            pltpu.sync_copy(
                x_hbm.at[pl.ds(in_base + c * IN_CHUNK, IN_CHUNK)], in_v
            )
            pltpu.sync_copy(
                out_v, out_hbm.at[pl.ds(out_base + c * OUT_CHUNK, OUT_CHUNK)]
            )
            return carry

        lax.fori_loop(0, N_CHUNKS // 2, chunk_body, 0)

    return k(xf)


def kernel(x):
    b, ch, _ = x.shape
    out = _resample_flat(x.reshape(-1))
    return out.reshape(b, ch, TOTAL_OUT // (b * ch))


# speedup vs baseline: 32.2428x; 31.8098x over previous
import functools

import jax
import jax.numpy as jnp
from jax import lax
from jax.experimental import pallas as pl
from jax.experimental.pallas import tpu as pltpu
from jax.experimental.pallas import tpu_sc as plsc

B, C, T_IN, T_OUT = 32, 2, 480000, 600000
T_OUT_PAD = 600064
OUT_CHUNK = 12_800
IN_CHUNK = OUT_CHUNK * 4 // 5
N_FULL = 46
TAIL_OUT = T_OUT_PAD - N_FULL * OUT_CHUNK
TAIL_IN_OFF = N_FULL * IN_CHUNK
TAIL_IN = T_IN - TAIL_IN_OFF
SUPER = OUT_CHUNK // 80
TAIL_VECS = TAIL_OUT // 16


def _resample(x):
    mesh = plsc.VectorSubcoreMesh(core_axis_name="c", subcore_axis_name="s")

    @functools.partial(
        pl.kernel,
        out_type=jax.ShapeDtypeStruct((B, C, T_OUT_PAD), jnp.float32),
        mesh=mesh,
        scratch_types=[
            pltpu.VMEM((C, IN_CHUNK), jnp.float32),
            pltpu.VMEM((C, IN_CHUNK), jnp.float32),
            pltpu.VMEM((C, OUT_CHUNK), jnp.float32),
            pltpu.VMEM((C, OUT_CHUNK), jnp.float32),
            pltpu.SemaphoreType.DMA((2,)),
            pltpu.SemaphoreType.DMA((2,)),
        ],
        compiler_params=pltpu.CompilerParams(
            needs_layout_passes=False,
            skip_device_barrier=True,
            disable_bounds_checks=True,
        ),
    )
    def k(x_hbm, out_hbm, in_v0, in_v1, out_v0, out_v1, in_sem, out_sem):
        in_bufs = (in_v0, in_v1)
        out_bufs = (out_v0, out_v1)
        wid = lax.axis_index("s") * 2 + lax.axis_index("c")

        lane = lax.iota(jnp.int32, 16)
        bases = [lax.div(4 * (16 * i + lane) + 2, 5) for i in range(5)]
        chans = [jnp.full((16,), ch, jnp.int32) for ch in range(C)]

        def in_copy(m, b):
            return pltpu.make_async_copy(
                x_hbm.at[wid, :, pl.ds(m * IN_CHUNK, IN_CHUNK)],
                in_bufs[b],
                in_sem.at[b],
            )

        def out_copy(m, b):
            return pltpu.make_async_copy(
                out_bufs[b],
                out_hbm.at[wid, :, pl.ds(m * OUT_CHUNK, OUT_CHUNK)],
                out_sem.at[b],
            )

        in_copy(0, 0).start()
        in_copy(1, 1).start()

        def outer(g, carry):
            for b in range(2):
                m = 2 * g + b
                in_copy(m, b).wait()

                @pl.when(g >= 1)
                def _():
                    out_copy(m - 2, b).wait()

                for ch in range(C):

                    @plsc.parallel_loop(0, SUPER, unroll=4)
                    def step(s):
                        off = 80 * s
                        shift = 64 * s
                        for i in range(5):
                            v = plsc.load_gather(
                                in_bufs[b], [chans[ch], bases[i] + shift]
                            )
                            out_bufs[b][ch, pl.ds(off + 16 * i, 16)] = v

                out_copy(m, b).start()

                @pl.when(g + 1 < N_FULL // 2)
                def _():
                    in_copy(m + 2, b).start()

            return carry

        lax.fori_loop(0, N_FULL // 2, outer, 0)

        out_copy(N_FULL - 2, 0).wait()
        tail_in = pltpu.make_async_copy(
            x_hbm.at[wid, :, pl.ds(TAIL_IN_OFF, TAIL_IN)],
            in_v0.at[:, pl.ds(0, TAIL_IN)],
            in_sem.at[0],
        )
        tail_in.start()
        tail_in.wait()

        for ch in range(C):

            @plsc.parallel_loop(0, TAIL_VECS, unroll=4)
            def tail_step(t):
                j = 16 * t + lane
                idx = jnp.minimum(lax.div(4 * j + 2, 5), TAIL_IN - 1)
                v = plsc.load_gather(in_v0, [chans[ch], idx])
                out_v0[ch, pl.ds(16 * t, 16)] = v

        out_copy(N_FULL - 1, 1).wait()
        tail_out = pltpu.make_async_copy(
            out_v0.at[:, pl.ds(0, TAIL_OUT)],
            out_hbm.at[wid, :, pl.ds(N_FULL * OUT_CHUNK, TAIL_OUT)],
            out_sem.at[0],
        )
        tail_out.start()
        tail_out.wait()

    return k(x)


def kernel(x):
    return _resample(x)[..., :T_OUT]


# speedup vs baseline: 32.9809x; 1.0229x over previous
import functools

import jax
import jax.numpy as jnp
from jax import lax
from jax.experimental import pallas as pl
from jax.experimental.pallas import tpu as pltpu
from jax.experimental.pallas import tpu_sc as plsc

B, C, T_IN, T_OUT = 32, 2, 480000, 600000
T_OUT_PAD = 600064
OUT_CHUNK = 11_520
IN_CHUNK = OUT_CHUNK * 4 // 5
N_FULL = 52
TAIL_OUT = T_OUT_PAD - N_FULL * OUT_CHUNK
TAIL_IN_OFF = N_FULL * IN_CHUNK
TAIL_IN = T_IN - TAIL_IN_OFF
SUPER = OUT_CHUNK // 80
TAIL_VECS = TAIL_OUT // 16


def _resample(x):
    mesh = plsc.VectorSubcoreMesh(core_axis_name="c", subcore_axis_name="s")

    @functools.partial(
        pl.kernel,
        out_type=jax.ShapeDtypeStruct((B, C, T_OUT_PAD), jnp.float32),
        mesh=mesh,
        scratch_types=[
            pltpu.VMEM((C, IN_CHUNK), jnp.float32),
            pltpu.VMEM((C, IN_CHUNK), jnp.float32),
            pltpu.VMEM((C, OUT_CHUNK), jnp.float32),
            pltpu.VMEM((C, OUT_CHUNK), jnp.float32),
            pltpu.VMEM((C, TAIL_IN), jnp.float32),
            pltpu.VMEM((C, TAIL_OUT), jnp.float32),
            pltpu.SemaphoreType.DMA((2,)),
            pltpu.SemaphoreType.DMA((2,)),
            pltpu.SemaphoreType.DMA,
            pltpu.SemaphoreType.DMA,
        ],
        compiler_params=pltpu.CompilerParams(
            needs_layout_passes=False,
            skip_device_barrier=True,
            disable_bounds_checks=True,
        ),
    )
    def k(x_hbm, out_hbm, in_v0, in_v1, out_v0, out_v1, tail_in_v,
          tail_out_v, in_sem, out_sem, tail_in_sem, tail_out_sem):
        in_bufs = (in_v0, in_v1)
        out_bufs = (out_v0, out_v1)
        wid = lax.axis_index("s") * 2 + lax.axis_index("c")

        lane = lax.iota(jnp.int32, 16)
        bases = [lax.div(4 * (16 * i + lane) + 2, 5) for i in range(5)]
        chans = [jnp.full((16,), ch, jnp.int32) for ch in range(C)]

        def in_copy(m, b):
            return pltpu.make_async_copy(
                x_hbm.at[wid, :, pl.ds(m * IN_CHUNK, IN_CHUNK)],
                in_bufs[b],
                in_sem.at[b],
            )

        def out_copy(m, b):
            return pltpu.make_async_copy(
                out_bufs[b],
                out_hbm.at[wid, :, pl.ds(m * OUT_CHUNK, OUT_CHUNK)],
                out_sem.at[b],
            )

        tail_in = pltpu.make_async_copy(
            x_hbm.at[wid, :, pl.ds(TAIL_IN_OFF, TAIL_IN)],
            tail_in_v,
            tail_in_sem,
        )
        tail_out = pltpu.make_async_copy(
            tail_out_v,
            out_hbm.at[wid, :, pl.ds(N_FULL * OUT_CHUNK, TAIL_OUT)],
            tail_out_sem,
        )

        tail_in.start()
        in_copy(0, 0).start()
        in_copy(1, 1).start()

        tail_in.wait()
        for ch in range(C):

            @plsc.parallel_loop(0, TAIL_VECS, unroll=4)
            def tail_step(t):
                j = 16 * t + lane
                idx = jnp.minimum(lax.div(4 * j + 2, 5), TAIL_IN - 1)
                v = plsc.load_gather(tail_in_v, [chans[ch], idx])
                tail_out_v[ch, pl.ds(16 * t, 16)] = v

        tail_out.start()

        def outer(g, carry):
            for b in range(2):
                m = 2 * g + b
                in_copy(m, b).wait()

                @pl.when(g >= 1)
                def _():
                    out_copy(m - 2, b).wait()

                for ch in range(C):

                    @plsc.parallel_loop(0, SUPER, unroll=4)
                    def step(s):
                        off = 80 * s
                        shift = 64 * s
                        for i in range(5):
                            v = plsc.load_gather(
                                in_bufs[b], [chans[ch], bases[i] + shift]
                            )
                            out_bufs[b][ch, pl.ds(off + 16 * i, 16)] = v

                out_copy(m, b).start()

                @pl.when(g + 1 < N_FULL // 2)
                def _():
                    in_copy(m + 2, b).start()

            return carry

        lax.fori_loop(0, N_FULL // 2, outer, 0)
        out_copy(N_FULL - 2, 0).wait()
        out_copy(N_FULL - 1, 1).wait()
        tail_out.wait()

    return k(x)


def kernel(x):
    return _resample(x)[..., :T_OUT]


# speedup vs baseline: 33.0002x; 1.0006x over previous
import functools

import jax
import jax.numpy as jnp
from jax import lax
from jax.experimental import pallas as pl
from jax.experimental.pallas import tpu as pltpu
from jax.experimental.pallas import tpu_sc as plsc

B, C, T_IN, T_OUT = 32, 2, 480000, 600000
T_OUT_PAD = 600064
OUT_CHUNK = 16_640
IN_CHUNK = OUT_CHUNK * 4 // 5
N_FULL = 36
TAIL_OUT = T_OUT_PAD - N_FULL * OUT_CHUNK
TAIL_IN_OFF = N_FULL * IN_CHUNK
TAIL_IN = T_IN - TAIL_IN_OFF
SUPER = OUT_CHUNK // 80
TAIL_VECS = TAIL_OUT // 16


def _resample(x):
    mesh = plsc.VectorSubcoreMesh(core_axis_name="c", subcore_axis_name="s")

    @functools.partial(
        pl.kernel,
        out_type=jax.ShapeDtypeStruct((B, C, T_OUT_PAD), jnp.float32),
        mesh=mesh,
        scratch_types=[
            pltpu.VMEM((C, IN_CHUNK), jnp.float32),
            pltpu.VMEM((C, IN_CHUNK), jnp.float32),
            pltpu.VMEM((C, OUT_CHUNK), jnp.float32),
            pltpu.VMEM((C, OUT_CHUNK), jnp.float32),
            pltpu.VMEM((C, TAIL_IN), jnp.float32),
            pltpu.VMEM((C, TAIL_OUT), jnp.float32),
            pltpu.SemaphoreType.DMA((2,)),
            pltpu.SemaphoreType.DMA((2,)),
            pltpu.SemaphoreType.DMA,
            pltpu.SemaphoreType.DMA,
        ],
        compiler_params=pltpu.CompilerParams(
            needs_layout_passes=False,
            skip_device_barrier=True,
            disable_bounds_checks=True,
        ),
    )
    def k(x_hbm, out_hbm, in_v0, in_v1, out_v0, out_v1, tail_in_v,
          tail_out_v, in_sem, out_sem, tail_in_sem, tail_out_sem):
        in_bufs = (in_v0, in_v1)
        out_bufs = (out_v0, out_v1)
        wid = lax.axis_index("s") * 2 + lax.axis_index("c")

        lane = lax.iota(jnp.int32, 16)
        bases = [lax.div(4 * (16 * i + lane) + 2, 5) for i in range(5)]
        chans = [jnp.full((16,), ch, jnp.int32) for ch in range(C)]

        def in_copy(m, b):
            return pltpu.make_async_copy(
                x_hbm.at[wid, :, pl.ds(m * IN_CHUNK, IN_CHUNK)],
                in_bufs[b],
                in_sem.at[b],
            )

        def out_copy(m, b):
            return pltpu.make_async_copy(
                out_bufs[b],
                out_hbm.at[wid, :, pl.ds(m * OUT_CHUNK, OUT_CHUNK)],
                out_sem.at[b],
            )

        tail_in = pltpu.make_async_copy(
            x_hbm.at[wid, :, pl.ds(TAIL_IN_OFF, TAIL_IN)],
            tail_in_v,
            tail_in_sem,
        )
        tail_out = pltpu.make_async_copy(
            tail_out_v,
            out_hbm.at[wid, :, pl.ds(N_FULL * OUT_CHUNK, TAIL_OUT)],
            tail_out_sem,
        )

        tail_in.start()
        in_copy(0, 0).start()
        in_copy(1, 1).start()

        tail_in.wait()
        for ch in range(C):

            @plsc.parallel_loop(0, TAIL_VECS, unroll=4)
            def tail_step(t):
                j = 16 * t + lane
                idx = jnp.minimum(lax.div(4 * j + 2, 5), TAIL_IN - 1)
                v = plsc.load_gather(tail_in_v, [chans[ch], idx])
                tail_out_v[ch, pl.ds(16 * t, 16)] = v

        tail_out.start()

        def outer(g, carry):
            for b in range(2):
                m = 2 * g + b
                in_copy(m, b).wait()

                @pl.when(g >= 1)
                def _():
                    out_copy(m - 2, b).wait()

                for ch in range(C):

                    @plsc.parallel_loop(0, SUPER, unroll=4)
                    def step(s):
                        off = 80 * s
                        shift = 64 * s
                        for i in range(5):
                            v = plsc.load_gather(
                                in_bufs[b], [chans[ch], bases[i] + shift]
                            )
                            out_bufs[b][ch, pl.ds(off + 16 * i, 16)] = v

                out_copy(m, b).start()

                @pl.when(g + 1 < N_FULL // 2)
                def _():
                    in_copy(m + 2, b).start()

            return carry

        lax.fori_loop(0, N_FULL // 2, outer, 0)
        out_copy(N_FULL - 2, 0).wait()
        out_copy(N_FULL - 1, 1).wait()
        tail_out.wait()

    return k(x)


def kernel(x):
    return _resample(x)[..., :T_OUT]
